# baseline (speedup 1.0000x reference)
import functools

import jax
import jax.numpy as jnp
import numpy as np
from jax import lax
from jax.experimental import pallas as pl
from jax.experimental.pallas import tpu as pltpu
from jax.experimental.pallas import tpu_sc as plsc

D_MODEL = 1024
B = 16
L = 512
R = 256
TROWS = B * (R + 1)
MROW = TROWS
NG = D_MODEL // 16


def _positional_encoding(d_model, seq_len):
    position = np.arange(seq_len, dtype=np.float32).reshape(-1, 1)
    i = np.arange(d_model) // 2
    exp_term = 2.0 * i / float(d_model)
    div_term = np.power(10000.0, exp_term).reshape(1, -1).astype(np.float32)
    pe = position / div_term
    pe[:, 0::2] = np.sin(pe[:, 0::2])
    pe[:, 1::2] = np.cos(pe[:, 1::2])
    return pe


_POS_NP = _positional_encoding(D_MODEL, L + 1)


@functools.partial(
    pl.kernel,
    mesh=plsc.VectorSubcoreMesh(core_axis_name="c", subcore_axis_name="s"),
    out_type=jax.ShapeDtypeStruct((B * (L + 1), D_MODEL), jnp.float32),
    scratch_types=[
        pltpu.VMEM((B * L,), jnp.int32),
        pltpu.VMEM((16, D_MODEL), jnp.float32),
        pltpu.VMEM((16,), jnp.int32),
        pltpu.VMEM((16,), jnp.int32),
        pltpu.VMEM((16, D_MODEL), jnp.float32),
        pltpu.VMEM((16, D_MODEL), jnp.float32),
        pltpu.VMEM((1, D_MODEL), jnp.float32),
        pltpu.SemaphoreType.DMA,
    ],
)
def _revert_sc(table, rv, pos1, pos0, out, rv_v, pos_v, idx_v,
               oidx_v, gbuf, zbuf, pbuf, sem):
    wid = lax.axis_index("s") * 2 + lax.axis_index("c")
    p0 = wid * 16

    pltpu.sync_copy(rv, rv_v)
    pltpu.sync_copy(pos1.at[pl.ds(p0, 16)], pos_v)

    lanes = jnp.arange(16, dtype=jnp.int32)

    @pl.when(wid < B)
    def _():
        pltpu.sync_copy(pos0, pbuf)
        idx_v[...] = jnp.full((16,), wid * (R + 1), jnp.int32)
        pltpu.async_copy(table.at[idx_v], zbuf, sem).wait()

        def add0(i, c):
            s = pl.ds(i * 16, 16)
            zbuf[0, s] = zbuf[0, s] + pbuf[0, s]
            return c

        lax.fori_loop(0, NG, add0, 0)
        oidx_v[...] = wid * (L + 1) + jnp.where(lanes > 0, p0 + lanes, 0)
        pltpu.async_copy(zbuf, out.at[oidx_v], sem).wait()

    for b in range(B):
        jj = rv_v[pl.ds(b * L + p0, 16)]
        idx_v[...] = jnp.where(jj < R, b * (R + 1) + jj + 1, MROW)
        pltpu.async_copy(table.at[idx_v], gbuf, sem).wait()

        def addp(i, c):
            s = pl.ds(i * 16, 16)
            for r in range(16):
                gbuf[r, s] = gbuf[r, s] + pos_v[r, s]
            return c

        lax.fori_loop(0, NG, addp, 0)
        oidx_v[...] = b * (L + 1) + 1 + p0 + lanes
        pltpu.async_copy(gbuf, out.at[oidx_v], sem).wait()


def kernel(data, mask_token, revert_idx, device, padding_mask):
    del device
    pm1 = jnp.concatenate(
        [jnp.ones((B, 1), dtype=padding_mask.dtype), padding_mask], axis=-1)
    remain = jnp.where(pm1[..., None] == 1, data, mask_token[None, None, :])
    table = jnp.concatenate(
        [remain.reshape(TROWS, D_MODEL), mask_token.reshape(1, D_MODEL)],
        axis=0)
    out2d = _revert_sc(table, revert_idx.reshape(-1),
                       jnp.asarray(_POS_NP[1:]), jnp.asarray(_POS_NP[0:1]))
    return out2d.reshape(B, L + 1, D_MODEL)
